# baseline (speedup 1.0000x reference)
import functools

import jax
import jax.numpy as jnp
import numpy as np
from jax import lax
from jax.experimental import pallas as pl
from jax.experimental.pallas import tpu as pltpu
from jax.experimental.pallas import tpu_sc as plsc

L_MAX = 3
Q = 32
PAIR = Q * Q
N_FEAT = 512
N_OUT = (L_MAX + 1) * PAIR
_OFFS = [0, 32, 128, 288]


def _sc_body(a_hbm, b_hbm, out_hbm, a_v, b_v, out_v):
    n_workers = 32
    per_w = a_hbm.shape[0] // n_workers
    wid = lax.axis_index("s") * 2 + lax.axis_index("c")
    base = wid * per_w

    def sample_body(i, carry):
        row = base + i
        pltpu.sync_copy(a_hbm.at[row], a_v)
        pltpu.sync_copy(b_hbm.at[row], b_v)
        for l in range(L_MAX + 1):
            ml = 2 * l + 1
            off = _OFFS[l]
            cg = np.float32(1.0 / np.sqrt(ml))
            b_lo = [b_v[pl.ds(off + m * Q, 16)] for m in range(ml)]
            b_hi = [b_v[pl.ds(off + m * Q + 16, 16)] for m in range(ml)]
            a_lo = [a_v[pl.ds(off + m * Q, 16)] for m in range(ml)]
            a_hi = [a_v[pl.ds(off + m * Q + 16, 16)] for m in range(ml)]
            for q in range(Q):
                acc0 = None
                acc1 = None
                for m in range(ml):
                    av = a_lo[m][q] if q < 16 else a_hi[m][q - 16]
                    t0 = av * b_lo[m]
                    t1 = av * b_hi[m]
                    acc0 = t0 if acc0 is None else acc0 + t0
                    acc1 = t1 if acc1 is None else acc1 + t1
                out_v[pl.ds(l * PAIR + q * Q, 16)] = acc0 * cg
                out_v[pl.ds(l * PAIR + q * Q + 16, 16)] = acc1 * cg
        pltpu.sync_copy(out_v, out_hbm.at[row])
        return carry

    lax.fori_loop(0, per_w, sample_body, 0)


@functools.partial(jax.jit, static_argnames=())
def kernel(density_nu_l0, density_nu_l1, density_nu_l2, density_nu_l3,
           density_1_l0, density_1_l1, density_1_l2, density_1_l3):
    n = density_nu_l0.shape[0]
    nus = (density_nu_l0, density_nu_l1, density_nu_l2, density_nu_l3)
    d1s = (density_1_l0, density_1_l1, density_1_l2, density_1_l3)
    a = jnp.concatenate([x.reshape(n, -1) for x in nus], axis=1)
    b = jnp.concatenate([x.reshape(n, -1) for x in d1s], axis=1)

    mesh = plsc.VectorSubcoreMesh(core_axis_name="c", subcore_axis_name="s")
    run = pl.kernel(
        _sc_body,
        out_type=jax.ShapeDtypeStruct((n, N_OUT), jnp.float32),
        mesh=mesh,
        scratch_types=[
            pltpu.VMEM((N_FEAT,), jnp.float32),
            pltpu.VMEM((N_FEAT,), jnp.float32),
            pltpu.VMEM((N_OUT,), jnp.float32),
        ],
    )
    return run(a, b)


# speedup vs baseline: 3.8700x; 3.8700x over previous
import functools

import jax
import jax.numpy as jnp
import numpy as np
from jax.experimental import pallas as pl

L_MAX = 3
Q = 32
PAIR = Q * Q
G = PAIR // 128


def _expansion_consts():
    rep = np.zeros((Q, PAIR), dtype=np.float32)
    tile = np.zeros((Q, 128), dtype=np.float32)
    for q in range(Q):
        rep[q, q * Q:(q + 1) * Q] = 1.0
        tile[q, q::Q] = 1.0
    return rep.astype(jnp.bfloat16), tile.astype(jnp.bfloat16)


_REP, _TILE = _expansion_consts()


def _ps_kernel(a0, a1, a2, a3, b0, b1, b2, b3, rep, tile, out_ref):
    a_refs = (a0, a1, a2, a3)
    b_refs = (b0, b1, b2, b3)
    for l in range(L_MAX + 1):
        ml = 2 * l + 1
        cg = np.float32(1.0 / np.sqrt(ml))
        a = a_refs[l][...]
        b = b_refs[l][...]
        rows = [None] * G
        for m in range(ml):
            am = a[:, m * Q:(m + 1) * Q]
            bm = b[:, m * Q:(m + 1) * Q]
            ar = jnp.dot(am, rep[...], preferred_element_type=jnp.float32)
            bt = jnp.dot(bm, tile[...], preferred_element_type=jnp.float32)
            for g in range(G):
                term = ar[:, g * 128:(g + 1) * 128] * bt
                rows[g] = term if rows[g] is None else rows[g] + term
        out_ref[:, l * PAIR:(l + 1) * PAIR] = jnp.concatenate(rows, axis=1) * cg


@functools.partial(jax.jit, static_argnames=())
def kernel(density_nu_l0, density_nu_l1, density_nu_l2, density_nu_l3,
           density_1_l0, density_1_l1, density_1_l2, density_1_l3):
    n = density_nu_l0.shape[0]
    sb = 256
    grid = (n // sb,)

    nus = (density_nu_l0, density_nu_l1, density_nu_l2, density_nu_l3)
    d1s = (density_1_l0, density_1_l1, density_1_l2, density_1_l3)
    abs_ = tuple(
        jnp.concatenate([nu, d1], axis=0).reshape(2 * n, -1).astype(jnp.bfloat16)
        for nu, d1 in zip(nus, d1s)
    )

    nblk = n // sb
    a_specs, b_specs = [], []
    for l in range(L_MAX + 1):
        a_specs.append(pl.BlockSpec((sb, (2 * l + 1) * Q), lambda i: (i, 0)))
        b_specs.append(
            pl.BlockSpec((sb, (2 * l + 1) * Q), lambda i, nb=nblk: (nb + i, 0)))
    in_specs = a_specs + b_specs
    in_specs.append(pl.BlockSpec(_REP.shape, lambda i: (0, 0)))
    in_specs.append(pl.BlockSpec(_TILE.shape, lambda i: (0, 0)))

    out = pl.pallas_call(
        _ps_kernel,
        grid=grid,
        in_specs=in_specs,
        out_specs=pl.BlockSpec((sb, (L_MAX + 1) * PAIR), lambda i: (i, 0)),
        out_shape=jax.ShapeDtypeStruct((n, (L_MAX + 1) * PAIR), jnp.float32),
    )(*abs_, *abs_, _REP, _TILE)
    return out


# speedup vs baseline: 4.0569x; 1.0483x over previous
import functools

import jax
import jax.numpy as jnp
import numpy as np
from jax.experimental import pallas as pl

L_MAX = 3
Q = 32
PAIR = Q * Q
G = PAIR // 128


def _expansion_consts():
    rep = np.zeros((Q, PAIR), dtype=np.float32)
    tile = np.zeros((Q, 128), dtype=np.float32)
    for q in range(Q):
        rep[q, q * Q:(q + 1) * Q] = 1.0
        tile[q, q::Q] = 1.0
    return rep.astype(jnp.bfloat16), tile.astype(jnp.bfloat16)


_REP, _TILE = _expansion_consts()


def _ps_kernel(a0, a1, a2, a3, b0, b1, b2, b3, rep, tile, out_ref):
    a_refs = (a0, a1, a2, a3)
    b_refs = (b0, b1, b2, b3)
    for l in range(L_MAX + 1):
        ml = 2 * l + 1
        cg = np.float32(1.0 / np.sqrt(ml))
        a = a_refs[l][...]
        b = b_refs[l][...]
        rows = [None] * G
        for m in range(ml):
            am = a[:, m * Q:(m + 1) * Q]
            bm = b[:, m * Q:(m + 1) * Q]
            ar = jnp.dot(am, rep[...], preferred_element_type=jnp.float32)
            bt = jnp.dot(bm, tile[...], preferred_element_type=jnp.float32)
            for g in range(G):
                term = ar[:, g * 128:(g + 1) * 128] * bt
                rows[g] = term if rows[g] is None else rows[g] + term
        out_ref[:, l * PAIR:(l + 1) * PAIR] = jnp.concatenate(rows, axis=1) * cg


@functools.partial(jax.jit, static_argnames=())
def kernel(density_nu_l0, density_nu_l1, density_nu_l2, density_nu_l3,
           density_1_l0, density_1_l1, density_1_l2, density_1_l3):
    n = density_nu_l0.shape[0]
    sb = 128
    grid = (n // sb,)

    nus = (density_nu_l0, density_nu_l1, density_nu_l2, density_nu_l3)
    d1s = (density_1_l0, density_1_l1, density_1_l2, density_1_l3)
    nus = tuple(x.reshape(n, -1).astype(jnp.bfloat16) for x in nus)
    d1s = tuple(x.reshape(n, -1).astype(jnp.bfloat16) for x in d1s)

    in_specs = []
    for l in range(L_MAX + 1):
        in_specs.append(pl.BlockSpec((sb, (2 * l + 1) * Q), lambda i: (i, 0)))
    in_specs = in_specs + in_specs
    in_specs.append(pl.BlockSpec(_REP.shape, lambda i: (0, 0)))
    in_specs.append(pl.BlockSpec(_TILE.shape, lambda i: (0, 0)))

    out = pl.pallas_call(
        _ps_kernel,
        grid=grid,
        in_specs=in_specs,
        out_specs=pl.BlockSpec((sb, (L_MAX + 1) * PAIR), lambda i: (i, 0)),
        out_shape=jax.ShapeDtypeStruct((n, (L_MAX + 1) * PAIR), jnp.float32),
    )(*nus, *d1s, _REP, _TILE)
    return out
